# Initial kernel scaffold; baseline (speedup 1.0000x reference)
import jax
import jax.numpy as jnp
from jax.experimental import pallas as pl


def kernel(x, edge_index, W1, b1, W2, b2):
    raise NotImplementedError("write your pallas kernel here")


# speedup vs baseline: 2.6500x; 2.6500x over previous
import functools

import jax
import jax.numpy as jnp
from jax import lax
from jax.experimental import pallas as pl
from jax.experimental.pallas import tpu as pltpu
from jax.experimental.pallas import tpu_sc as plsc

NC = 2
NS = 16
NW = NC * NS
CH = 80
NQ = 4
Q = 32

_SC_PARAMS = pltpu.CompilerParams(use_tc_tiling_on_sc=False)


def _sc_mesh():
    return plsc.VectorSubcoreMesh(core_axis_name="c", subcore_axis_name="s")


def _degree_kernel(np_, e):
    epw = e // NW
    nch = epw // CH
    rps = np_ // NS

    @functools.partial(
        pl.kernel,
        mesh=_sc_mesh(),
        compiler_params=_SC_PARAMS,
        out_type=jax.ShapeDtypeStruct((NC * np_, 16), jnp.float32),
        scratch_types=[
            pltpu.VMEM((CH,), jnp.int32),
            pltpu.VMEM((CH, 16), jnp.float32),
            pltpu.VMEM_SHARED((np_, 16), jnp.float32),
        ],
    )
    def deg(row_hbm, ones_hbm, zrow_hbm, out_hbm, ridx, ones_v, acc):
        c = lax.axis_index("c")
        s = lax.axis_index("s")
        pltpu.sync_copy(zrow_hbm, acc.at[pl.ds(s * rps, rps)])
        pltpu.sync_copy(ones_hbm, ones_v)
        plsc.subcore_barrier()
        base0 = (c * NS + s) * epw

        def body(g, carry):
            pltpu.sync_copy(row_hbm.at[pl.ds(base0 + g * CH, CH)], ridx)
            pltpu.sync_copy(ones_v, acc.at[ridx], add=True)
            return carry

        lax.fori_loop(0, nch, body, 0)
        plsc.subcore_barrier()
        pltpu.sync_copy(acc.at[pl.ds(s * rps, rps)],
                        out_hbm.at[pl.ds(c * np_ + s * rps, rps)])

    return deg


def _spmv_kernel(np_, e):
    epw = e // NW
    nch = epw // CH
    rps = np_ // NS

    @functools.partial(
        pl.kernel,
        mesh=_sc_mesh(),
        compiler_params=_SC_PARAMS,
        out_type=jax.ShapeDtypeStruct((NC * NQ * np_, Q), jnp.float32),
        scratch_types=[
            pltpu.VMEM((CH,), jnp.int32),
            pltpu.VMEM((CH,), jnp.int32),
            pltpu.VMEM((CH, Q), jnp.float32),
            pltpu.VMEM_SHARED((np_, Q), jnp.float32),
            pltpu.SemaphoreType.DMA,
        ],
    )
    def spmv(u0_hbm, u1_hbm, u2_hbm, u3_hbm, row_hbm, col_hbm, zrow_hbm,
             out_hbm, ridx, cidx, rows, acc, sem):
        c = lax.axis_index("c")
        s = lax.axis_index("s")
        base0 = (c * NS + s) * epw

        for q, u_hbm in enumerate((u0_hbm, u1_hbm, u2_hbm, u3_hbm)):
            pltpu.sync_copy(zrow_hbm, acc.at[pl.ds(s * rps, rps)])
            plsc.subcore_barrier()

            def body(g, carry):
                base = base0 + g * CH
                pltpu.sync_copy(row_hbm.at[pl.ds(base, CH)], ridx)
                pltpu.sync_copy(col_hbm.at[pl.ds(base, CH)], cidx)
                pltpu.async_copy(u_hbm.at[ridx], rows, sem).wait()
                pltpu.sync_copy(rows, acc.at[cidx], add=True)
                return carry

            lax.fori_loop(0, nch, body, 0)
            plsc.subcore_barrier()
            pltpu.sync_copy(
                acc.at[pl.ds(s * rps, rps)],
                out_hbm.at[pl.ds((c * NQ + q) * np_ + s * rps, rps)])
            plsc.subcore_barrier()

    return spmv


def _full_spec(shape):
    nd = len(shape)
    return pl.BlockSpec(shape, lambda i, _nd=nd: (0,) * _nd)


def _tc_call(body, out_widths, bn, np_, d, *args):
    grid = np_ // bn
    in_specs = []
    for a in args:
        if a.ndim == 4:
            in_specs.append(
                pl.BlockSpec((NC, NQ, bn, Q), lambda i: (0, 0, i, 0)))
        elif a.shape[0] == np_:
            in_specs.append(
                pl.BlockSpec((bn, a.shape[1]), lambda i: (i, 0)))
        else:
            in_specs.append(_full_spec(a.shape))
    out_shapes = tuple(
        jax.ShapeDtypeStruct((np_, w), jnp.float32) for w in out_widths)
    out_specs = tuple(
        pl.BlockSpec((bn, w), lambda i: (i, 0)) for w in out_widths)
    return pl.pallas_call(
        body,
        grid=(grid,),
        in_specs=in_specs,
        out_shape=out_shapes,
        out_specs=out_specs,
    )(*args)


def _sum_parts(sref):
    return jnp.concatenate(
        [sref[0, q] + sref[1, q] for q in range(NQ)], axis=1)


def _store_quarters(u, urefs):
    for q, uref in enumerate(urefs):
        uref[...] = u[:, q * Q:(q + 1) * Q]


def _tc_a(dparts, x, w0, bn, np_, d):
    def body(dref, xref, wref, dis16_ref, uq0, uq1, uq2, uq3, acc_ref):
        deg = dref[0, :, 0:1] + dref[1, :, 0:1]
        dis = jnp.where(deg > 0.0, lax.rsqrt(jnp.maximum(deg, 1e-30)), 0.0)
        dis16_ref[...] = jnp.broadcast_to(dis, (bn, 16))
        xb = xref[...]
        _store_quarters(dis * xb, (uq0, uq1, uq2, uq3))
        acc_ref[...] = jnp.dot(xb, wref[...],
                               preferred_element_type=jnp.float32)

    grid = np_ // bn
    in_specs = [
        pl.BlockSpec((NC, bn, 16), lambda i: (0, i, 0)),
        pl.BlockSpec((bn, d), lambda i: (i, 0)),
        _full_spec(w0.shape),
    ]
    out_shapes = tuple(
        jax.ShapeDtypeStruct((np_, w), jnp.float32)
        for w in (16, Q, Q, Q, Q, d))
    out_specs = tuple(
        pl.BlockSpec((bn, w), lambda i: (i, 0)) for w in (16, Q, Q, Q, Q, d))
    return pl.pallas_call(
        body, grid=(grid,), in_specs=in_specs,
        out_shape=out_shapes, out_specs=out_specs)(dparts, x, w0)


def _tc_b(sparts, dis16, wk, acc, bn, np_, d):
    def body(sref, dref, wref, aref, uq0, uq1, uq2, uq3, acc_ref):
        dis = dref[:, 0:1]
        tx1 = -dis * _sum_parts(sref)
        _store_quarters(dis * tx1, (uq0, uq1, uq2, uq3))
        acc_ref[...] = aref[...] + jnp.dot(
            tx1, wref[...], preferred_element_type=jnp.float32)
    return _tc_call(body, (Q, Q, Q, Q, d), bn, np_, d,
                    sparts, dis16, wk, acc)


def _tc_c(sparts, dis16, tx0, acc, wk, b, w0n, bn, np_, d):
    def body(sref, dref, tref, aref, wref, bref, wnref,
             h_ref, uq0, uq1, uq2, uq3, acc_ref):
        dis = dref[:, 0:1]
        tx2 = -2.0 * dis * _sum_parts(sref) - tref[...]
        pre = aref[...] + jnp.dot(tx2, wref[...],
                                  preferred_element_type=jnp.float32)
        h = jnp.maximum(pre + bref[...], 0.0)
        h_ref[...] = h
        _store_quarters(dis * h, (uq0, uq1, uq2, uq3))
        acc_ref[...] = jnp.dot(h, wnref[...],
                               preferred_element_type=jnp.float32)
    return _tc_call(body, (d, Q, Q, Q, Q, d), bn, np_, d,
                    sparts, dis16, tx0, acc, wk, b, w0n)


def _tc_e(sparts, dis16, tx0, acc, wk, b, bn, np_, d):
    def body(sref, dref, tref, aref, wref, bref, out_ref):
        dis = dref[:, 0:1]
        tx2 = -2.0 * dis * _sum_parts(sref) - tref[...]
        out_ref[...] = aref[...] + jnp.dot(
            tx2, wref[...], preferred_element_type=jnp.float32) + bref[...]
    return _tc_call(body, (d,), bn, np_, d, sparts, dis16, tx0, acc, wk, b)[0]


def kernel(x, edge_index, W1, b1, W2, b2):
    n, d = x.shape
    e = edge_index.shape[1]
    np_ = 10240
    bn = 1024
    rps = np_ // NS
    row = edge_index[0]
    col = edge_index[1]

    xp = jnp.pad(x, ((0, np_ - n), (0, 0)))
    zrow_q = jnp.zeros((rps, Q), jnp.float32)
    zrow_16 = jnp.zeros((rps, 16), jnp.float32)
    ones16 = jnp.ones((CH, 16), jnp.float32)
    b1r = jnp.reshape(b1, (1, d))
    b2r = jnp.reshape(b2, (1, d))

    deg_fn = _degree_kernel(np_, e)
    spmv_fn = _spmv_kernel(np_, e)

    def spmv(uq):
        parts = spmv_fn(uq[0], uq[1], uq[2], uq[3], row, col, zrow_q)
        return jnp.reshape(parts, (NC, NQ, np_, Q))

    dparts = jnp.reshape(deg_fn(row, ones16, zrow_16), (NC, np_, 16))
    dis16, u0a, u0b_, u0c, u0d, acc = _tc_a(dparts, xp, W1[0], bn, np_, d)

    s1 = spmv((u0a, u0b_, u0c, u0d))
    uq1 = _tc_b(s1, dis16, W1[1], acc, bn, np_, d)
    acc = uq1[-1]

    s2 = spmv(uq1[:4])
    h, v0, v1, v2, v3, acc2 = _tc_c(s2, dis16, xp, acc, W1[2], b1r, W2[0],
                                    bn, np_, d)

    s1b = spmv((v0, v1, v2, v3))
    uq2 = _tc_b(s1b, dis16, W2[1], acc2, bn, np_, d)
    acc2 = uq2[-1]

    s2b = spmv(uq2[:4])
    out = _tc_e(s2b, dis16, h, acc2, W2[2], b2r, bn, np_, d)

    return out[:n]


# speedup vs baseline: 7.6691x; 2.8941x over previous
import functools

import jax
import jax.numpy as jnp
from jax import lax
from jax.experimental import pallas as pl
from jax.experimental.pallas import tpu as pltpu
from jax.experimental.pallas import tpu_sc as plsc

NC = 2
NS = 16
NW = NC * NS
CH = 80
NQ = 4
Q = 32

_SC_PARAMS = pltpu.CompilerParams(use_tc_tiling_on_sc=False)


def _sc_mesh():
    return plsc.VectorSubcoreMesh(core_axis_name="c", subcore_axis_name="s")


def _degree_kernel(np_, e):
    epw = e // NW
    nch = epw // CH
    rps = np_ // NS
    assert nch % 5 == 0

    @functools.partial(
        pl.kernel,
        mesh=_sc_mesh(),
        compiler_params=_SC_PARAMS,
        out_type=jax.ShapeDtypeStruct((NC * np_, 16), jnp.float32),
        scratch_types=[
            pltpu.VMEM((nch, CH), jnp.int32),
            pltpu.VMEM((CH, 16), jnp.float32),
            pltpu.VMEM_SHARED((np_, 16), jnp.float32),
            pltpu.SemaphoreType.DMA,
        ],
    )
    def deg(row2_hbm, ones_hbm, zrow_hbm, out_hbm, ridx2, ones_v, acc, sem):
        c = lax.axis_index("c")
        s = lax.axis_index("s")
        w = c * NS + s
        pltpu.sync_copy(row2_hbm.at[pl.ds(w * nch, nch)], ridx2)
        pltpu.sync_copy(zrow_hbm, acc.at[pl.ds(s * rps, rps)])
        pltpu.sync_copy(ones_hbm, ones_v)
        plsc.subcore_barrier()

        def body(k, carry):
            for j in range(5):
                pltpu.async_copy(ones_v, acc.at[ridx2.at[5 * k + j]], sem,
                                 add=True)
            for j in range(5):
                pltpu.make_async_copy(ones_v, acc.at[ridx2.at[5 * k + j]],
                                      sem).wait()
            return carry

        lax.fori_loop(0, nch // 5, body, 0)
        plsc.subcore_barrier()
        pltpu.sync_copy(acc.at[pl.ds(s * rps, rps)],
                        out_hbm.at[pl.ds(c * np_ + s * rps, rps)])

    return deg


def _spmv_kernel(np_, e):
    epw = e // NW
    nch = epw // CH
    rps = np_ // NS
    assert nch % 2 == 1

    @functools.partial(
        pl.kernel,
        mesh=_sc_mesh(),
        compiler_params=_SC_PARAMS,
        out_type=jax.ShapeDtypeStruct((NC * NQ * np_, Q), jnp.float32),
        scratch_types=[
            pltpu.VMEM((nch, CH), jnp.int32),
            pltpu.VMEM((nch, CH), jnp.int32),
            pltpu.VMEM((CH, Q), jnp.float32),
            pltpu.VMEM((CH, Q), jnp.float32),
            pltpu.VMEM_SHARED((np_, Q), jnp.float32),
            pltpu.SemaphoreType.DMA,
            pltpu.SemaphoreType.DMA,
        ],
    )
    def spmv(u0_hbm, u1_hbm, u2_hbm, u3_hbm, row2_hbm, col2_hbm, zrow_hbm,
             out_hbm, ridx2, cidx2, buf0, buf1, acc, sem0, sem1):
        c = lax.axis_index("c")
        s = lax.axis_index("s")
        w = c * NS + s
        pltpu.sync_copy(row2_hbm.at[pl.ds(w * nch, nch)], ridx2)
        pltpu.sync_copy(col2_hbm.at[pl.ds(w * nch, nch)], cidx2)
        bufs = (buf0, buf1)
        sems = (sem0, sem1)

        for q, u_hbm in enumerate((u0_hbm, u1_hbm, u2_hbm, u3_hbm)):
            pltpu.sync_copy(zrow_hbm, acc.at[pl.ds(s * rps, rps)])
            plsc.subcore_barrier()

            def start(g, b, u=u_hbm):
                pltpu.async_copy(u.at[ridx2.at[g]], bufs[b], sems[b])

            def wait(g, b, u=u_hbm):
                pltpu.make_async_copy(u.at[ridx2.at[g]], bufs[b],
                                      sems[b]).wait()

            def scat(g, b):
                pltpu.sync_copy(bufs[b], acc.at[cidx2.at[g]], add=True)

            start(0, 0)

            def body(k, carry):
                g = 2 * k
                start(g + 1, 1)
                wait(g, 0)
                scat(g, 0)
                start(g + 2, 0)
                wait(g + 1, 1)
                scat(g + 1, 1)
                return carry

            lax.fori_loop(0, (nch - 1) // 2, body, 0)
            wait(nch - 1, 0)
            scat(nch - 1, 0)
            plsc.subcore_barrier()
            pltpu.sync_copy(
                acc.at[pl.ds(s * rps, rps)],
                out_hbm.at[pl.ds((c * NQ + q) * np_ + s * rps, rps)])
            plsc.subcore_barrier()

    return spmv


def _full_spec(shape):
    nd = len(shape)
    return pl.BlockSpec(shape, lambda i, _nd=nd: (0,) * _nd)


def _tc_call(body, out_widths, bn, np_, d, *args):
    grid = np_ // bn
    in_specs = []
    for a in args:
        if a.ndim == 4:
            in_specs.append(
                pl.BlockSpec((NC, NQ, bn, Q), lambda i: (0, 0, i, 0)))
        elif a.shape[0] == np_:
            in_specs.append(
                pl.BlockSpec((bn, a.shape[1]), lambda i: (i, 0)))
        else:
            in_specs.append(_full_spec(a.shape))
    out_shapes = tuple(
        jax.ShapeDtypeStruct((np_, w), jnp.float32) for w in out_widths)
    out_specs = tuple(
        pl.BlockSpec((bn, w), lambda i: (i, 0)) for w in out_widths)
    return pl.pallas_call(
        body,
        grid=(grid,),
        in_specs=in_specs,
        out_shape=out_shapes,
        out_specs=out_specs,
    )(*args)


def _sum_parts(sref):
    return jnp.concatenate(
        [sref[0, q] + sref[1, q] for q in range(NQ)], axis=1)


def _store_quarters(u, urefs):
    for q, uref in enumerate(urefs):
        uref[...] = u[:, q * Q:(q + 1) * Q]


def _tc_a(dparts, x, w0, bn, np_, d):
    def body(dref, xref, wref, dis16_ref, uq0, uq1, uq2, uq3, acc_ref):
        deg = dref[0, :, 0:1] + dref[1, :, 0:1]
        dis = jnp.where(deg > 0.0, lax.rsqrt(jnp.maximum(deg, 1e-30)), 0.0)
        dis16_ref[...] = jnp.broadcast_to(dis, (bn, 16))
        xb = xref[...]
        _store_quarters(dis * xb, (uq0, uq1, uq2, uq3))
        acc_ref[...] = jnp.dot(xb, wref[...],
                               preferred_element_type=jnp.float32)

    grid = np_ // bn
    in_specs = [
        pl.BlockSpec((NC, bn, 16), lambda i: (0, i, 0)),
        pl.BlockSpec((bn, d), lambda i: (i, 0)),
        _full_spec(w0.shape),
    ]
    out_shapes = tuple(
        jax.ShapeDtypeStruct((np_, w), jnp.float32)
        for w in (16, Q, Q, Q, Q, d))
    out_specs = tuple(
        pl.BlockSpec((bn, w), lambda i: (i, 0)) for w in (16, Q, Q, Q, Q, d))
    return pl.pallas_call(
        body, grid=(grid,), in_specs=in_specs,
        out_shape=out_shapes, out_specs=out_specs)(dparts, x, w0)


def _tc_b(sparts, dis16, wk, acc, bn, np_, d):
    def body(sref, dref, wref, aref, uq0, uq1, uq2, uq3, acc_ref):
        dis = dref[:, 0:1]
        tx1 = -dis * _sum_parts(sref)
        _store_quarters(dis * tx1, (uq0, uq1, uq2, uq3))
        acc_ref[...] = aref[...] + jnp.dot(
            tx1, wref[...], preferred_element_type=jnp.float32)
    return _tc_call(body, (Q, Q, Q, Q, d), bn, np_, d,
                    sparts, dis16, wk, acc)


def _tc_c(sparts, dis16, tx0, acc, wk, b, w0n, bn, np_, d):
    def body(sref, dref, tref, aref, wref, bref, wnref,
             h_ref, uq0, uq1, uq2, uq3, acc_ref):
        dis = dref[:, 0:1]
        tx2 = -2.0 * dis * _sum_parts(sref) - tref[...]
        pre = aref[...] + jnp.dot(tx2, wref[...],
                                  preferred_element_type=jnp.float32)
        h = jnp.maximum(pre + bref[...], 0.0)
        h_ref[...] = h
        _store_quarters(dis * h, (uq0, uq1, uq2, uq3))
        acc_ref[...] = jnp.dot(h, wnref[...],
                               preferred_element_type=jnp.float32)
    return _tc_call(body, (d, Q, Q, Q, Q, d), bn, np_, d,
                    sparts, dis16, tx0, acc, wk, b, w0n)


def _tc_e(sparts, dis16, tx0, acc, wk, b, bn, np_, d):
    def body(sref, dref, tref, aref, wref, bref, out_ref):
        dis = dref[:, 0:1]
        tx2 = -2.0 * dis * _sum_parts(sref) - tref[...]
        out_ref[...] = aref[...] + jnp.dot(
            tx2, wref[...], preferred_element_type=jnp.float32) + bref[...]
    return _tc_call(body, (d,), bn, np_, d, sparts, dis16, tx0, acc, wk, b)[0]


def kernel(x, edge_index, W1, b1, W2, b2):
    n, d = x.shape
    e = edge_index.shape[1]
    np_ = 10240
    bn = 1024
    rps = np_ // NS
    row2 = jnp.reshape(edge_index[0], (e // CH, CH))
    col2 = jnp.reshape(edge_index[1], (e // CH, CH))

    xp = jnp.pad(x, ((0, np_ - n), (0, 0)))
    zrow_q = jnp.zeros((rps, Q), jnp.float32)
    zrow_16 = jnp.zeros((rps, 16), jnp.float32)
    ones16 = jnp.ones((CH, 16), jnp.float32)
    b1r = jnp.reshape(b1, (1, d))
    b2r = jnp.reshape(b2, (1, d))

    deg_fn = _degree_kernel(np_, e)
    spmv_fn = _spmv_kernel(np_, e)

    def spmv(uq):
        parts = spmv_fn(uq[0], uq[1], uq[2], uq[3], row2, col2, zrow_q)
        return jnp.reshape(parts, (NC, NQ, np_, Q))

    dparts = jnp.reshape(deg_fn(row2, ones16, zrow_16), (NC, np_, 16))
    dis16, u0a, u0b_, u0c, u0d, acc = _tc_a(dparts, xp, W1[0], bn, np_, d)

    s1 = spmv((u0a, u0b_, u0c, u0d))
    uq1 = _tc_b(s1, dis16, W1[1], acc, bn, np_, d)
    acc = uq1[-1]

    s2 = spmv(uq1[:4])
    h, v0, v1, v2, v3, acc2 = _tc_c(s2, dis16, xp, acc, W1[2], b1r, W2[0],
                                    bn, np_, d)

    s1b = spmv((v0, v1, v2, v3))
    uq2 = _tc_b(s1b, dis16, W2[1], acc2, bn, np_, d)
    acc2 = uq2[-1]

    s2b = spmv(uq2[:4])
    out = _tc_e(s2b, dis16, h, acc2, W2[2], b2r, bn, np_, d)

    return out[:n]


# speedup vs baseline: 10.2573x; 1.3375x over previous
import functools

import jax
import jax.numpy as jnp
from jax import lax
from jax.experimental import pallas as pl
from jax.experimental.pallas import tpu as pltpu
from jax.experimental.pallas import tpu_sc as plsc

NC = 2
NS = 16
NW = NC * NS
CH = 80
NQ = 4
Q = 32

_SC_PARAMS = pltpu.CompilerParams(use_tc_tiling_on_sc=False)


def _sc_mesh():
    return plsc.VectorSubcoreMesh(core_axis_name="c", subcore_axis_name="s")


def _degree_kernel(np_, e):
    epw = e // NW
    nch = epw // CH
    rps = np_ // NS

    @functools.partial(
        pl.kernel,
        mesh=_sc_mesh(),
        compiler_params=_SC_PARAMS,
        out_type=jax.ShapeDtypeStruct((NC * np_, 16), jnp.float32),
        scratch_types=[
            pltpu.VMEM((CH,), jnp.int32),
            pltpu.VMEM((CH,), jnp.int32),
            pltpu.VMEM((CH, 16), jnp.float32),
            pltpu.VMEM_SHARED((np_, 16), jnp.float32),
            pltpu.SemaphoreType.DMA,
        ],
    )
    def deg(row2_hbm, ones_hbm, zrow_hbm, out_hbm, ridx0, ridx1, ones_v,
            acc, sem):
        c = lax.axis_index("c")
        s = lax.axis_index("s")
        w = c * NS + s
        pltpu.sync_copy(zrow_hbm, acc.at[pl.ds(s * rps, rps)])
        pltpu.sync_copy(ones_hbm, ones_v)
        plsc.subcore_barrier()
        row_flat = row2_hbm

        def body(k, carry):
            g = 2 * k
            pltpu.sync_copy(row_flat.at[w * nch + g], ridx0)
            pltpu.async_copy(ones_v, acc.at[ridx0], sem, add=True)
            pltpu.sync_copy(row_flat.at[w * nch + g + 1], ridx1)
            pltpu.async_copy(ones_v, acc.at[ridx1], sem, add=True)
            pltpu.make_async_copy(ones_v, acc.at[ridx0], sem).wait()
            pltpu.make_async_copy(ones_v, acc.at[ridx1], sem).wait()
            return carry

        lax.fori_loop(0, nch // 2, body, 0)
        if nch % 2 == 1:
            pltpu.sync_copy(row_flat.at[w * nch + nch - 1], ridx0)
            pltpu.sync_copy(ones_v, acc.at[ridx0], add=True)
        plsc.subcore_barrier()
        pltpu.sync_copy(acc.at[pl.ds(s * rps, rps)],
                        out_hbm.at[pl.ds(c * np_ + s * rps, rps)])

    return deg


def _spmv_kernel(np_, e):
    epw = e // NW
    nch = epw // CH
    rps = np_ // NS
    nk = (nch - 1) // 4
    assert nch == 4 * nk + 1

    @functools.partial(
        pl.kernel,
        mesh=_sc_mesh(),
        compiler_params=_SC_PARAMS,
        out_type=jax.ShapeDtypeStruct((NC * NQ * np_, Q), jnp.float32),
        scratch_types=[
            pltpu.VMEM((nch, CH), jnp.int32),
            pltpu.VMEM((nch, CH), jnp.int32),
            [pltpu.VMEM((CH, Q), jnp.float32) for _ in range(4)],
            pltpu.VMEM_SHARED((np_, Q), jnp.float32),
            [pltpu.SemaphoreType.DMA for _ in range(4)],
            [pltpu.SemaphoreType.DMA for _ in range(4)],
        ],
    )
    def spmv(u0_hbm, u1_hbm, u2_hbm, u3_hbm, row2_hbm, col2_hbm, zrow_hbm,
             out_hbm, ridx2, cidx2, bufs, acc, gsems, ssems):
        c = lax.axis_index("c")
        s = lax.axis_index("s")
        w = c * NS + s
        pltpu.sync_copy(row2_hbm.at[pl.ds(w * nch, nch)], ridx2)
        pltpu.sync_copy(col2_hbm.at[pl.ds(w * nch, nch)], cidx2)

        for q, u_hbm in enumerate((u0_hbm, u1_hbm, u2_hbm, u3_hbm)):
            pltpu.sync_copy(zrow_hbm, acc.at[pl.ds(s * rps, rps)])
            plsc.subcore_barrier()

            def start_g(g, b, u=u_hbm):
                pltpu.async_copy(u.at[ridx2.at[g]], bufs[b], gsems[b])

            def wait_g(g, b, u=u_hbm):
                pltpu.make_async_copy(u.at[ridx2.at[g]], bufs[b],
                                      gsems[b]).wait()

            def start_s(g, b):
                pltpu.async_copy(bufs[b], acc.at[cidx2.at[g]], ssems[b],
                                 add=True)

            def wait_s(g, b):
                pltpu.make_async_copy(bufs[b], acc.at[cidx2.at[g]],
                                      ssems[b]).wait()

            for b in range(3):
                start_g(b, b)
            for j in range(4):
                if j > 0:
                    wait_s(j - 1, (j + 3) % 4)
                start_g(j + 3, (j + 3) % 4)
                wait_g(j, j)
                start_s(j, j)

            def body(k, carry):
                for j in range(4):
                    g = 4 * k + j
                    bi = (j + 3) % 4
                    wait_s(g - 1, bi)
                    start_g(g + 3, bi)
                    wait_g(g, j)
                    start_s(g, j)
                return carry

            lax.fori_loop(1, nk - 1, body, 0)

            g0 = 4 * (nk - 1)
            for j in range(4):
                bi = (j + 3) % 4
                wait_s(g0 + j - 1, bi)
                if g0 + j + 3 <= nch - 1:
                    start_g(g0 + j + 3, bi)
                wait_g(g0 + j, j)
                start_s(g0 + j, j)

            wait_g(nch - 1, 0)
            start_s(nch - 1, 0)
            wait_s(nch - 2, 3)
            wait_s(nch - 1, 0)
            plsc.subcore_barrier()
            pltpu.sync_copy(
                acc.at[pl.ds(s * rps, rps)],
                out_hbm.at[pl.ds((c * NQ + q) * np_ + s * rps, rps)])
            plsc.subcore_barrier()

    return spmv


def _full_spec(shape):
    nd = len(shape)
    return pl.BlockSpec(shape, lambda i, _nd=nd: (0,) * _nd)


def _tc_call(body, out_widths, bn, np_, d, *args):
    grid = np_ // bn
    in_specs = []
    for a in args:
        if a.ndim == 4:
            in_specs.append(
                pl.BlockSpec((NC, NQ, bn, Q), lambda i: (0, 0, i, 0)))
        elif a.shape[0] == np_:
            in_specs.append(
                pl.BlockSpec((bn, a.shape[1]), lambda i: (i, 0)))
        else:
            in_specs.append(_full_spec(a.shape))
    out_shapes = tuple(
        jax.ShapeDtypeStruct((np_, w), jnp.float32) for w in out_widths)
    out_specs = tuple(
        pl.BlockSpec((bn, w), lambda i: (i, 0)) for w in out_widths)
    return pl.pallas_call(
        body,
        grid=(grid,),
        in_specs=in_specs,
        out_shape=out_shapes,
        out_specs=out_specs,
    )(*args)


def _sum_parts(sref):
    return jnp.concatenate(
        [sref[0, q] + sref[1, q] for q in range(NQ)], axis=1)


def _store_quarters(u, urefs):
    for q, uref in enumerate(urefs):
        uref[...] = u[:, q * Q:(q + 1) * Q]


def _tc_a(dparts, x, w0, bn, np_, d):
    def body(dref, xref, wref, dis16_ref, uq0, uq1, uq2, uq3, acc_ref):
        deg = dref[0, :, 0:1] + dref[1, :, 0:1]
        dis = jnp.where(deg > 0.0, lax.rsqrt(jnp.maximum(deg, 1e-30)), 0.0)
        dis16_ref[...] = jnp.broadcast_to(dis, (bn, 16))
        xb = xref[...]
        _store_quarters(dis * xb, (uq0, uq1, uq2, uq3))
        acc_ref[...] = jnp.dot(xb, wref[...],
                               preferred_element_type=jnp.float32)

    grid = np_ // bn
    in_specs = [
        pl.BlockSpec((NC, bn, 16), lambda i: (0, i, 0)),
        pl.BlockSpec((bn, d), lambda i: (i, 0)),
        _full_spec(w0.shape),
    ]
    out_shapes = tuple(
        jax.ShapeDtypeStruct((np_, w), jnp.float32)
        for w in (16, Q, Q, Q, Q, d))
    out_specs = tuple(
        pl.BlockSpec((bn, w), lambda i: (i, 0)) for w in (16, Q, Q, Q, Q, d))
    return pl.pallas_call(
        body, grid=(grid,), in_specs=in_specs,
        out_shape=out_shapes, out_specs=out_specs)(dparts, x, w0)


def _tc_b(sparts, dis16, wk, acc, bn, np_, d):
    def body(sref, dref, wref, aref, uq0, uq1, uq2, uq3, acc_ref):
        dis = dref[:, 0:1]
        tx1 = -dis * _sum_parts(sref)
        _store_quarters(dis * tx1, (uq0, uq1, uq2, uq3))
        acc_ref[...] = aref[...] + jnp.dot(
            tx1, wref[...], preferred_element_type=jnp.float32)
    return _tc_call(body, (Q, Q, Q, Q, d), bn, np_, d,
                    sparts, dis16, wk, acc)


def _tc_c(sparts, dis16, tx0, acc, wk, b, w0n, bn, np_, d):
    def body(sref, dref, tref, aref, wref, bref, wnref,
             h_ref, uq0, uq1, uq2, uq3, acc_ref):
        dis = dref[:, 0:1]
        tx2 = -2.0 * dis * _sum_parts(sref) - tref[...]
        pre = aref[...] + jnp.dot(tx2, wref[...],
                                  preferred_element_type=jnp.float32)
        h = jnp.maximum(pre + bref[...], 0.0)
        h_ref[...] = h
        _store_quarters(dis * h, (uq0, uq1, uq2, uq3))
        acc_ref[...] = jnp.dot(h, wnref[...],
                               preferred_element_type=jnp.float32)
    return _tc_call(body, (d, Q, Q, Q, Q, d), bn, np_, d,
                    sparts, dis16, tx0, acc, wk, b, w0n)


def _tc_e(sparts, dis16, tx0, acc, wk, b, bn, np_, d):
    def body(sref, dref, tref, aref, wref, bref, out_ref):
        dis = dref[:, 0:1]
        tx2 = -2.0 * dis * _sum_parts(sref) - tref[...]
        out_ref[...] = aref[...] + jnp.dot(
            tx2, wref[...], preferred_element_type=jnp.float32) + bref[...]
    return _tc_call(body, (d,), bn, np_, d, sparts, dis16, tx0, acc, wk, b)[0]


def kernel(x, edge_index, W1, b1, W2, b2):
    n, d = x.shape
    e = edge_index.shape[1]
    np_ = 10240
    bn = 1024
    rps = np_ // NS
    row2 = jnp.reshape(edge_index[0], (e // CH, CH))
    col2 = jnp.reshape(edge_index[1], (e // CH, CH))

    xp = jnp.pad(x, ((0, np_ - n), (0, 0)))
    zrow_q = jnp.zeros((rps, Q), jnp.float32)
    zrow_16 = jnp.zeros((rps, 16), jnp.float32)
    ones16 = jnp.ones((CH, 16), jnp.float32)
    b1r = jnp.reshape(b1, (1, d))
    b2r = jnp.reshape(b2, (1, d))

    deg_fn = _degree_kernel(np_, e)
    spmv_fn = _spmv_kernel(np_, e)

    def spmv(uq):
        parts = spmv_fn(uq[0], uq[1], uq[2], uq[3], row2, col2, zrow_q)
        return jnp.reshape(parts, (NC, NQ, np_, Q))

    dparts = jnp.reshape(deg_fn(row2, ones16, zrow_16), (NC, np_, 16))
    dis16, u0a, u0b_, u0c, u0d, acc = _tc_a(dparts, xp, W1[0], bn, np_, d)

    s1 = spmv((u0a, u0b_, u0c, u0d))
    uq1 = _tc_b(s1, dis16, W1[1], acc, bn, np_, d)
    acc = uq1[-1]

    s2 = spmv(uq1[:4])
    h, v0, v1, v2, v3, acc2 = _tc_c(s2, dis16, xp, acc, W1[2], b1r, W2[0],
                                    bn, np_, d)

    s1b = spmv((v0, v1, v2, v3))
    uq2 = _tc_b(s1b, dis16, W2[1], acc2, bn, np_, d)
    acc2 = uq2[-1]

    s2b = spmv(uq2[:4])
    out = _tc_e(s2b, dis16, h, acc2, W2[2], b2r, bn, np_, d)

    return out[:n]


# speedup vs baseline: 13.8757x; 1.3528x over previous
import functools

import jax
import jax.numpy as jnp
from jax import lax
from jax.experimental import pallas as pl
from jax.experimental.pallas import tpu as pltpu
from jax.experimental.pallas import tpu_sc as plsc

NC = 2
NS = 16
NW = NC * NS
CH = 80
NQ = 2
Q = 64

_SC_PARAMS = pltpu.CompilerParams(use_tc_tiling_on_sc=False)


def _sc_mesh():
    return plsc.VectorSubcoreMesh(core_axis_name="c", subcore_axis_name="s")


def _degree_kernel(np_, e):
    epw = e // NW
    nch = epw // CH
    rps = np_ // NS

    @functools.partial(
        pl.kernel,
        mesh=_sc_mesh(),
        compiler_params=_SC_PARAMS,
        out_type=jax.ShapeDtypeStruct((NC * np_, 16), jnp.float32),
        scratch_types=[
            pltpu.VMEM((CH,), jnp.int32),
            pltpu.VMEM((CH,), jnp.int32),
            pltpu.VMEM((CH, 16), jnp.float32),
            pltpu.VMEM_SHARED((np_, 16), jnp.float32),
            pltpu.SemaphoreType.DMA,
        ],
    )
    def deg(row2_hbm, ones_hbm, zrow_hbm, out_hbm, ridx0, ridx1, ones_v,
            acc, sem):
        c = lax.axis_index("c")
        s = lax.axis_index("s")
        w = c * NS + s
        pltpu.sync_copy(zrow_hbm, acc.at[pl.ds(s * rps, rps)])
        pltpu.sync_copy(ones_hbm, ones_v)
        plsc.subcore_barrier()
        row_flat = row2_hbm

        def body(k, carry):
            g = 2 * k
            pltpu.sync_copy(row_flat.at[w * nch + g], ridx0)
            pltpu.async_copy(ones_v, acc.at[ridx0], sem, add=True)
            pltpu.sync_copy(row_flat.at[w * nch + g + 1], ridx1)
            pltpu.async_copy(ones_v, acc.at[ridx1], sem, add=True)
            pltpu.make_async_copy(ones_v, acc.at[ridx0], sem).wait()
            pltpu.make_async_copy(ones_v, acc.at[ridx1], sem).wait()
            return carry

        lax.fori_loop(0, nch // 2, body, 0)
        if nch % 2 == 1:
            pltpu.sync_copy(row_flat.at[w * nch + nch - 1], ridx0)
            pltpu.sync_copy(ones_v, acc.at[ridx0], add=True)
        plsc.subcore_barrier()
        pltpu.sync_copy(acc.at[pl.ds(s * rps, rps)],
                        out_hbm.at[pl.ds(c * np_ + s * rps, rps)])

    return deg


def _spmv_kernel(np_, e):
    epw = e // NW
    nch = epw // CH
    rps = np_ // NS
    nk = (nch - 1) // 4
    assert nch == 4 * nk + 1

    @functools.partial(
        pl.kernel,
        mesh=_sc_mesh(),
        compiler_params=_SC_PARAMS,
        out_type=jax.ShapeDtypeStruct((NC * NQ * np_, Q), jnp.float32),
        scratch_types=[
            pltpu.VMEM((nch, CH), jnp.int32),
            pltpu.VMEM((nch, CH), jnp.int32),
            [pltpu.VMEM((CH, Q), jnp.float32) for _ in range(4)],
            pltpu.VMEM_SHARED((np_, Q), jnp.float32),
            [pltpu.SemaphoreType.DMA for _ in range(4)],
            [pltpu.SemaphoreType.DMA for _ in range(4)],
        ],
    )
    def spmv(u0_hbm, u1_hbm, row2_hbm, col2_hbm, zrow_hbm,
             out_hbm, ridx2, cidx2, bufs, acc, gsems, ssems):
        c = lax.axis_index("c")
        s = lax.axis_index("s")
        w = c * NS + s
        pltpu.sync_copy(row2_hbm.at[pl.ds(w * nch, nch)], ridx2)
        pltpu.sync_copy(col2_hbm.at[pl.ds(w * nch, nch)], cidx2)

        for q, u_hbm in enumerate((u0_hbm, u1_hbm)):
            pltpu.sync_copy(zrow_hbm, acc.at[pl.ds(s * rps, rps)])
            plsc.subcore_barrier()

            def start_g(g, b, u=u_hbm):
                pltpu.async_copy(u.at[ridx2.at[g]], bufs[b], gsems[b])

            def wait_g(g, b, u=u_hbm):
                pltpu.make_async_copy(u.at[ridx2.at[g]], bufs[b],
                                      gsems[b]).wait()

            def start_s(g, b):
                pltpu.async_copy(bufs[b], acc.at[cidx2.at[g]], ssems[b],
                                 add=True)

            def wait_s(g, b):
                pltpu.make_async_copy(bufs[b], acc.at[cidx2.at[g]],
                                      ssems[b]).wait()

            for b in range(3):
                start_g(b, b)
            for j in range(4):
                if j > 0:
                    wait_s(j - 1, (j + 3) % 4)
                start_g(j + 3, (j + 3) % 4)
                wait_g(j, j)
                start_s(j, j)

            def body(k, carry):
                for j in range(4):
                    g = 4 * k + j
                    bi = (j + 3) % 4
                    wait_s(g - 1, bi)
                    start_g(g + 3, bi)
                    wait_g(g, j)
                    start_s(g, j)
                return carry

            lax.fori_loop(1, nk - 1, body, 0)

            g0 = 4 * (nk - 1)
            for j in range(4):
                bi = (j + 3) % 4
                wait_s(g0 + j - 1, bi)
                if g0 + j + 3 <= nch - 1:
                    start_g(g0 + j + 3, bi)
                wait_g(g0 + j, j)
                start_s(g0 + j, j)

            wait_g(nch - 1, 0)
            start_s(nch - 1, 0)
            wait_s(nch - 2, 3)
            wait_s(nch - 1, 0)
            plsc.subcore_barrier()
            pltpu.sync_copy(
                acc.at[pl.ds(s * rps, rps)],
                out_hbm.at[pl.ds((c * NQ + q) * np_ + s * rps, rps)])
            plsc.subcore_barrier()

    return spmv


def _full_spec(shape):
    nd = len(shape)
    return pl.BlockSpec(shape, lambda i, _nd=nd: (0,) * _nd)


def _tc_call(body, out_widths, bn, np_, d, *args):
    grid = np_ // bn
    in_specs = []
    for a in args:
        if a.ndim == 4:
            in_specs.append(
                pl.BlockSpec((NC, NQ, bn, Q), lambda i: (0, 0, i, 0)))
        elif a.shape[0] == np_:
            in_specs.append(
                pl.BlockSpec((bn, a.shape[1]), lambda i: (i, 0)))
        else:
            in_specs.append(_full_spec(a.shape))
    out_shapes = tuple(
        jax.ShapeDtypeStruct((np_, w), jnp.float32) for w in out_widths)
    out_specs = tuple(
        pl.BlockSpec((bn, w), lambda i: (i, 0)) for w in out_widths)
    return pl.pallas_call(
        body,
        grid=(grid,),
        in_specs=in_specs,
        out_shape=out_shapes,
        out_specs=out_specs,
    )(*args)


def _sum_parts(sref):
    return jnp.concatenate(
        [sref[0, q] + sref[1, q] for q in range(NQ)], axis=1)


def _store_halves(u, urefs):
    for q, uref in enumerate(urefs):
        uref[...] = u[:, q * Q:(q + 1) * Q]


def _tc_dis(dparts, bn, np_):
    def body(dref, dis16_ref):
        deg = dref[0, :, 0:1] + dref[1, :, 0:1]
        dis = jnp.where(deg > 0.0, lax.rsqrt(jnp.maximum(deg, 1e-30)), 0.0)
        dis16_ref[...] = jnp.broadcast_to(dis, (bn, 16))

    return pl.pallas_call(
        body,
        grid=(np_ // bn,),
        in_specs=[pl.BlockSpec((NC, bn, 16), lambda i: (0, i, 0))],
        out_shape=jax.ShapeDtypeStruct((np_, 16), jnp.float32),
        out_specs=pl.BlockSpec((bn, 16), lambda i: (i, 0)),
    )(dparts)


def _tc_a(dis16, x, w0, bn, np_, d):
    def body(dref, xref, wref, uq0, uq1, acc_ref):
        dis = dref[:, 0:1]
        xb = xref[...]
        _store_halves(dis * xb, (uq0, uq1))
        acc_ref[...] = jnp.dot(xb, wref[...],
                               preferred_element_type=jnp.float32)
    return _tc_call(body, (Q, Q, d), bn, np_, d, dis16, x, w0)


def _tc_b(sparts, dis16, wk, acc, bn, np_, d):
    def body(sref, dref, wref, aref, uq0, uq1, acc_ref):
        dis = dref[:, 0:1]
        tx1 = -dis * _sum_parts(sref)
        _store_halves(dis * tx1, (uq0, uq1))
        acc_ref[...] = aref[...] + jnp.dot(
            tx1, wref[...], preferred_element_type=jnp.float32)
    return _tc_call(body, (Q, Q, d), bn, np_, d, sparts, dis16, wk, acc)


def _tc_e(sparts, dis16, tx0, acc, wk, b, rfl, bn, np_, d):
    def body(sref, dref, tref, aref, wref, bref, rref, out_ref):
        dis = dref[:, 0:1]
        tx2 = -2.0 * dis * _sum_parts(sref) - tref[...]
        o = aref[...] + jnp.dot(
            tx2, wref[...], preferred_element_type=jnp.float32) + bref[...]
        out_ref[...] = jnp.where(rref[...] > 0.0, jnp.maximum(o, 0.0), o)
    return _tc_call(body, (d,), bn, np_, d,
                    sparts, dis16, tx0, acc, wk, b, rfl)[0]


def kernel(x, edge_index, W1, b1, W2, b2):
    n, d = x.shape
    e = edge_index.shape[1]
    np_ = 10240
    bn = 1024
    rps = np_ // NS
    row2 = jnp.reshape(edge_index[0], (e // CH, CH))
    col2 = jnp.reshape(edge_index[1], (e // CH, CH))

    xp = jnp.pad(x, ((0, np_ - n), (0, 0)))
    zrow_q = jnp.zeros((rps, Q), jnp.float32)
    zrow_16 = jnp.zeros((rps, 16), jnp.float32)
    ones16 = jnp.ones((CH, 16), jnp.float32)
    ws = jnp.stack((W1, W2))
    bs = jnp.stack((jnp.reshape(b1, (1, d)), jnp.reshape(b2, (1, d))))
    rfls = jnp.stack((jnp.ones((1, d), jnp.float32),
                      jnp.zeros((1, d), jnp.float32)))

    deg_fn = _degree_kernel(np_, e)
    spmv_fn = _spmv_kernel(np_, e)

    def spmv(u0, u1):
        parts = spmv_fn(u0, u1, row2, col2, zrow_q)
        return jnp.reshape(parts, (NC, NQ, np_, Q))

    dparts = jnp.reshape(deg_fn(row2, ones16, zrow_16), (NC, np_, 16))
    dis16 = _tc_dis(dparts, bn, np_)

    def layer(x_in, wbr):
        wk, bk, rfl = wbr
        u0, u1, acc = _tc_a(dis16, x_in, wk[0], bn, np_, d)
        s1 = spmv(u0, u1)
        v0, v1, acc = _tc_b(s1, dis16, wk[1], acc, bn, np_, d)
        s2 = spmv(v0, v1)
        out = _tc_e(s2, dis16, x_in, acc, wk[2], bk, rfl, bn, np_, d)
        return out, None

    out, _ = lax.scan(layer, xp, (ws, bs, rfls))
    return out[:n]


# speedup vs baseline: 14.9313x; 1.0761x over previous
import functools

import jax
import jax.numpy as jnp
from jax import lax
from jax.experimental import pallas as pl
from jax.experimental.pallas import tpu as pltpu
from jax.experimental.pallas import tpu_sc as plsc

NC = 2
NS = 16
NW = NC * NS
CH = 80
NQ = 2
Q = 64

_SC_PARAMS = pltpu.CompilerParams(use_tc_tiling_on_sc=False)


def _sc_mesh():
    return plsc.VectorSubcoreMesh(core_axis_name="c", subcore_axis_name="s")


def _degree_kernel(np_, e):
    epw = e // NW
    nch = epw // CH
    rps = np_ // NS
    assert nch % 5 == 0

    @functools.partial(
        pl.kernel,
        mesh=_sc_mesh(),
        compiler_params=_SC_PARAMS,
        out_type=jax.ShapeDtypeStruct((NC * np_, 16), jnp.float32),
        scratch_types=[
            pltpu.VMEM((nch, CH), jnp.int32),
            pltpu.VMEM((CH, 16), jnp.float32),
            pltpu.VMEM_SHARED((np_, 16), jnp.float32),
            pltpu.SemaphoreType.DMA,
        ],
    )
    def deg(row2_hbm, ones_hbm, zrow_hbm, out_hbm, ridx2, ones_v, acc, sem):
        c = lax.axis_index("c")
        s = lax.axis_index("s")
        w = c * NS + s
        pltpu.sync_copy(row2_hbm.at[pl.ds(w * nch, nch)], ridx2)
        pltpu.sync_copy(zrow_hbm, acc.at[pl.ds(s * rps, rps)])
        pltpu.sync_copy(ones_hbm, ones_v)
        plsc.subcore_barrier()

        def body(k, carry):
            for j in range(5):
                pltpu.async_copy(ones_v, acc.at[ridx2.at[5 * k + j]], sem,
                                 add=True)
            for j in range(5):
                pltpu.make_async_copy(ones_v, acc.at[ridx2.at[5 * k + j]],
                                      sem).wait()
            return carry

        lax.fori_loop(0, nch // 5, body, 0)
        plsc.subcore_barrier()
        pltpu.sync_copy(acc.at[pl.ds(s * rps, rps)],
                        out_hbm.at[pl.ds(c * np_ + s * rps, rps)])

    return deg


def _spmv_kernel(np_, e):
    epw = e // NW
    nch = epw // CH
    rps = np_ // NS
    nk = (nch - 1) // 4
    assert nch == 4 * nk + 1

    @functools.partial(
        pl.kernel,
        mesh=_sc_mesh(),
        compiler_params=_SC_PARAMS,
        out_type=jax.ShapeDtypeStruct((NC * NQ * np_, Q), jnp.float32),
        scratch_types=[
            pltpu.VMEM((nch, CH), jnp.int32),
            pltpu.VMEM((nch, CH), jnp.int32),
            [pltpu.VMEM((CH, Q), jnp.float32) for _ in range(4)],
            pltpu.VMEM_SHARED((np_, Q), jnp.float32),
            [pltpu.SemaphoreType.DMA for _ in range(4)],
            [pltpu.SemaphoreType.DMA for _ in range(4)],
        ],
    )
    def spmv(u0_hbm, u1_hbm, row2_hbm, col2_hbm, zrow_hbm,
             out_hbm, ridx2, cidx2, bufs, acc, gsems, ssems):
        c = lax.axis_index("c")
        s = lax.axis_index("s")
        w = c * NS + s
        pltpu.sync_copy(row2_hbm.at[pl.ds(w * nch, nch)], ridx2)
        pltpu.sync_copy(col2_hbm.at[pl.ds(w * nch, nch)], cidx2)

        for q, u_hbm in enumerate((u0_hbm, u1_hbm)):
            pltpu.sync_copy(zrow_hbm, acc.at[pl.ds(s * rps, rps)])
            plsc.subcore_barrier()

            def start_g(g, b, u=u_hbm):
                pltpu.async_copy(u.at[ridx2.at[g]], bufs[b], gsems[b])

            def wait_g(g, b, u=u_hbm):
                pltpu.make_async_copy(u.at[ridx2.at[g]], bufs[b],
                                      gsems[b]).wait()

            def start_s(g, b):
                pltpu.async_copy(bufs[b], acc.at[cidx2.at[g]], ssems[b],
                                 add=True)

            def wait_s(g, b):
                pltpu.make_async_copy(bufs[b], acc.at[cidx2.at[g]],
                                      ssems[b]).wait()

            for b in range(3):
                start_g(b, b)
            for j in range(4):
                if j > 0:
                    wait_s(j - 1, (j + 3) % 4)
                start_g(j + 3, (j + 3) % 4)
                wait_g(j, j)
                start_s(j, j)

            def body(k, carry):
                for j in range(4):
                    g = 4 * k + j
                    bi = (j + 3) % 4
                    wait_s(g - 1, bi)
                    start_g(g + 3, bi)
                    wait_g(g, j)
                    start_s(g, j)
                return carry

            lax.fori_loop(1, nk - 1, body, 0)

            g0 = 4 * (nk - 1)
            for j in range(4):
                bi = (j + 3) % 4
                wait_s(g0 + j - 1, bi)
                if g0 + j + 3 <= nch - 1:
                    start_g(g0 + j + 3, bi)
                wait_g(g0 + j, j)
                start_s(g0 + j, j)

            wait_g(nch - 1, 0)
            start_s(nch - 1, 0)
            wait_s(nch - 2, 3)
            wait_s(nch - 1, 0)
            plsc.subcore_barrier()
            pltpu.sync_copy(
                acc.at[pl.ds(s * rps, rps)],
                out_hbm.at[pl.ds((c * NQ + q) * np_ + s * rps, rps)])
            plsc.subcore_barrier()

    return spmv


def _full_spec(shape):
    nd = len(shape)
    return pl.BlockSpec(shape, lambda i, _nd=nd: (0,) * _nd)


def _tc_call(body, out_widths, bn, np_, d, *args):
    grid = np_ // bn
    in_specs = []
    for a in args:
        if a.ndim == 4:
            in_specs.append(
                pl.BlockSpec((NC, NQ, bn, Q), lambda i: (0, 0, i, 0)))
        elif a.shape[0] == np_:
            in_specs.append(
                pl.BlockSpec((bn, a.shape[1]), lambda i: (i, 0)))
        else:
            in_specs.append(_full_spec(a.shape))
    out_shapes = tuple(
        jax.ShapeDtypeStruct((np_, w), jnp.float32) for w in out_widths)
    out_specs = tuple(
        pl.BlockSpec((bn, w), lambda i: (i, 0)) for w in out_widths)
    return pl.pallas_call(
        body,
        grid=(grid,),
        in_specs=in_specs,
        out_shape=out_shapes,
        out_specs=out_specs,
    )(*args)


def _sum_parts(sref):
    return jnp.concatenate(
        [sref[0, q] + sref[1, q] for q in range(NQ)], axis=1)


def _store_halves(u, urefs):
    for q, uref in enumerate(urefs):
        uref[...] = u[:, q * Q:(q + 1) * Q]


def _tc_dis(dparts, bn, np_):
    def body(dref, dis16_ref):
        deg = dref[0, :, 0:1] + dref[1, :, 0:1]
        dis = jnp.where(deg > 0.0, lax.rsqrt(jnp.maximum(deg, 1e-30)), 0.0)
        dis16_ref[...] = jnp.broadcast_to(dis, (bn, 16))

    return pl.pallas_call(
        body,
        grid=(np_ // bn,),
        in_specs=[pl.BlockSpec((NC, bn, 16), lambda i: (0, i, 0))],
        out_shape=jax.ShapeDtypeStruct((np_, 16), jnp.float32),
        out_specs=pl.BlockSpec((bn, 16), lambda i: (i, 0)),
    )(dparts)


def _tc_a(dis16, x, w0, bn, np_, d):
    def body(dref, xref, wref, uq0, uq1, acc_ref):
        dis = dref[:, 0:1]
        xb = xref[...]
        _store_halves(dis * xb, (uq0, uq1))
        acc_ref[...] = jnp.dot(xb, wref[...],
                               preferred_element_type=jnp.float32)
    return _tc_call(body, (Q, Q, d), bn, np_, d, dis16, x, w0)


def _tc_b(sparts, dis16, wk, acc, bn, np_, d):
    def body(sref, dref, wref, aref, uq0, uq1, acc_ref):
        dis = dref[:, 0:1]
        tx1 = -dis * _sum_parts(sref)
        _store_halves(dis * tx1, (uq0, uq1))
        acc_ref[...] = aref[...] + jnp.dot(
            tx1, wref[...], preferred_element_type=jnp.float32)
    return _tc_call(body, (Q, Q, d), bn, np_, d, sparts, dis16, wk, acc)


def _tc_e(sparts, dis16, tx0, acc, wk, b, rfl, bn, np_, d):
    def body(sref, dref, tref, aref, wref, bref, rref, out_ref):
        dis = dref[:, 0:1]
        tx2 = -2.0 * dis * _sum_parts(sref) - tref[...]
        o = aref[...] + jnp.dot(
            tx2, wref[...], preferred_element_type=jnp.float32) + bref[...]
        out_ref[...] = jnp.where(rref[...] > 0.0, jnp.maximum(o, 0.0), o)
    return _tc_call(body, (d,), bn, np_, d,
                    sparts, dis16, tx0, acc, wk, b, rfl)[0]


def kernel(x, edge_index, W1, b1, W2, b2):
    n, d = x.shape
    e = edge_index.shape[1]
    np_ = 10240
    bn = 1024
    rps = np_ // NS
    row2 = jnp.reshape(edge_index[0], (e // CH, CH))
    col2 = jnp.reshape(edge_index[1], (e // CH, CH))

    xp = jnp.pad(x, ((0, np_ - n), (0, 0)))
    zrow_q = jnp.zeros((rps, Q), jnp.float32)
    zrow_16 = jnp.zeros((rps, 16), jnp.float32)
    ones16 = jnp.ones((CH, 16), jnp.float32)
    ws = jnp.stack((W1, W2))
    bs = jnp.stack((jnp.reshape(b1, (1, d)), jnp.reshape(b2, (1, d))))
    rfls = jnp.stack((jnp.ones((1, d), jnp.float32),
                      jnp.zeros((1, d), jnp.float32)))

    deg_fn = _degree_kernel(np_, e)
    spmv_fn = _spmv_kernel(np_, e)

    def spmv(u0, u1):
        parts = spmv_fn(u0, u1, row2, col2, zrow_q)
        return jnp.reshape(parts, (NC, NQ, np_, Q))

    dparts = jnp.reshape(deg_fn(row2, ones16, zrow_16), (NC, np_, 16))
    dis16 = _tc_dis(dparts, bn, np_)

    def layer(x_in, wbr):
        wk, bk, rfl = wbr
        u0, u1, acc = _tc_a(dis16, x_in, wk[0], bn, np_, d)
        s1 = spmv(u0, u1)
        v0, v1, acc = _tc_b(s1, dis16, wk[1], acc, bn, np_, d)
        s2 = spmv(v0, v1)
        out = _tc_e(s2, dis16, x_in, acc, wk[2], bk, rfl, bn, np_, d)
        return out, None

    out, _ = lax.scan(layer, xp, (ws, bs, rfls))
    return out[:n]
